# baseline (speedup 1.0000x reference)
import functools

import jax
import jax.numpy as jnp
from jax import lax
from jax.experimental import pallas as pl
from jax.experimental.pallas import tpu as pltpu
from jax.experimental.pallas import tpu_sc as plsc

_N_CLASSES = 1000000
_EMB_DIM = 32
_BATCH = 16384

_NUM_CORES = 2
_NUM_SUBCORES = 16
_NW = _NUM_CORES * _NUM_SUBCORES
_B_PER_W = _BATCH // _NW
_CHUNK = 128
_NCHUNK = _B_PER_W // _CHUNK


_mesh = plsc.VectorSubcoreMesh(core_axis_name="c", subcore_axis_name="s")


@functools.partial(
    pl.kernel,
    mesh=_mesh,
    out_type=jax.ShapeDtypeStruct((_BATCH, _EMB_DIM), jnp.float32),
    scratch_types=[
        pltpu.VMEM((_NCHUNK, _CHUNK), jnp.int32),
        pltpu.VMEM((_B_PER_W, _EMB_DIM), jnp.float32),
        pltpu.SemaphoreType.DMA,
    ],
    compiler_params=pltpu.CompilerParams(use_tc_tiling_on_sc=False),
)
def _gather_kernel(idx_hbm, table_hbm, out_hbm, idx_v, rows_v, sem):
    wid = lax.axis_index("s") * _NUM_CORES + lax.axis_index("c")
    pltpu.sync_copy(idx_hbm.at[wid], idx_v)
    copies = []
    for j in range(_NCHUNK):
        copies.append(
            pltpu.async_copy(
                table_hbm.at[idx_v.at[j]],
                rows_v.at[pl.ds(j * _CHUNK, _CHUNK)],
                sem,
            )
        )
    for c in copies:
        c.wait()
    pltpu.sync_copy(rows_v, out_hbm.at[pl.ds(wid * _B_PER_W, _B_PER_W)])


def kernel(targets, embs):
    idx = targets.astype(jnp.int32).reshape(_NW, _NCHUNK, _CHUNK)
    return _gather_kernel(idx, embs)


# speedup vs baseline: 1.6560x; 1.6560x over previous
import functools

import jax
import jax.numpy as jnp
from jax import lax
from jax.experimental import pallas as pl
from jax.experimental.pallas import tpu as pltpu
from jax.experimental.pallas import tpu_sc as plsc

_N_CLASSES = 1000000
_EMB_DIM = 32
_BATCH = 16384

_NUM_CORES = 2
_NUM_SUBCORES = 16
_NW = _NUM_CORES * _NUM_SUBCORES
_B_PER_W = _BATCH // _NW


_mesh = plsc.VectorSubcoreMesh(core_axis_name="c", subcore_axis_name="s")


@functools.partial(
    pl.kernel,
    mesh=_mesh,
    out_type=jax.ShapeDtypeStruct((_BATCH, _EMB_DIM), jnp.float32),
    scratch_types=[
        pltpu.VMEM((_B_PER_W,), jnp.int32),
        pltpu.VMEM((_B_PER_W, _EMB_DIM), jnp.float32),
        pltpu.SemaphoreType.DMA,
    ],
)
def _gather_kernel(idx_hbm, table_hbm, out_hbm, idx_v, rows_v, sem):
    wid = lax.axis_index("s") * _NUM_CORES + lax.axis_index("c")
    base = wid * _B_PER_W
    pltpu.sync_copy(idx_hbm.at[pl.ds(base, _B_PER_W)], idx_v)

    def body(g, carry):
        vec = idx_v[pl.ds(g * 16, 16)]
        for l in range(16):
            t = vec[l]
            i = g * 16 + l
            pltpu.async_copy(
                table_hbm.at[pl.ds(t, 1)], rows_v.at[pl.ds(i, 1)], sem)
        return carry

    lax.fori_loop(0, _B_PER_W // 16, body, 0)
    pltpu.make_async_copy(
        table_hbm.at[pl.ds(0, _B_PER_W)], rows_v, sem).wait()
    pltpu.sync_copy(rows_v, out_hbm.at[pl.ds(base, _B_PER_W)])


def kernel(targets, embs):
    return _gather_kernel(targets.astype(jnp.int32), embs)


# speedup vs baseline: 1.6570x; 1.0006x over previous
import functools

import jax
import jax.numpy as jnp
from jax import lax
from jax.experimental import pallas as pl
from jax.experimental.pallas import tpu as pltpu
from jax.experimental.pallas import tpu_sc as plsc

_N_CLASSES = 1000000
_EMB_DIM = 32
_BATCH = 16384
_SLOTS = 250000
_SLOT_W = 128

_NUM_CORES = 2
_NUM_SUBCORES = 16
_NW = _NUM_CORES * _NUM_SUBCORES
_B_PER_W = _BATCH // _NW
_CHUNK = 128
_NCHUNK = _B_PER_W // _CHUNK


_mesh = plsc.VectorSubcoreMesh(core_axis_name="c", subcore_axis_name="s")


@functools.partial(
    pl.kernel,
    mesh=_mesh,
    out_type=jax.ShapeDtypeStruct((_BATCH, _EMB_DIM), jnp.float32),
    scratch_types=[
        pltpu.VMEM((_B_PER_W,), jnp.int32),
        pltpu.VMEM((_B_PER_W, _EMB_DIM), jnp.float32),
        pltpu.SemaphoreType.DMA,
    ],
    compiler_params=pltpu.CompilerParams(skip_device_barrier=True),
)
def _gather_kernel(idx_hbm, table_hbm, out_hbm, idx_v, rows_v, sem):
    wid = lax.axis_index("s") * _NUM_CORES + lax.axis_index("c")
    base = wid * _B_PER_W
    pltpu.sync_copy(idx_hbm.at[pl.ds(base, _B_PER_W)], idx_v)

    def body(g, carry):
        vec = idx_v[pl.ds(g * 16, 16)]
        for l in range(16):
            t = vec[l]
            i = g * 16 + l
            pltpu.async_copy(
                table_hbm.at[pl.ds(t, 1)], rows_v.at[pl.ds(i, 1)], sem)
        return carry

    lax.fori_loop(0, _B_PER_W // 16, body, 0)
    pltpu.make_async_copy(
        table_hbm.at[pl.ds(0, _B_PER_W)], rows_v, sem).wait()
    pltpu.sync_copy(rows_v, out_hbm.at[pl.ds(base, _B_PER_W)])


def kernel(targets, embs):
    return _gather_kernel(targets.astype(jnp.int32), embs)


# speedup vs baseline: 3.1665x; 1.9110x over previous
import functools

import jax
import jax.numpy as jnp
from jax import lax
from jax.experimental import pallas as pl
from jax.experimental.pallas import tpu as pltpu
from jax.experimental.pallas import tpu_sc as plsc

_N_CLASSES = 1000000
_EMB_DIM = 32
_BATCH = 16384
_LANES = 128

_NUM_CORES = 2
_NUM_SUBCORES = 16
_NW = _NUM_CORES * _NUM_SUBCORES
_B_PER_W = _BATCH // _NW
_RING = 8
_ROUNDS = _B_PER_W // _RING


_mesh = plsc.VectorSubcoreMesh(core_axis_name="c", subcore_axis_name="s")


@functools.partial(
    pl.kernel,
    mesh=_mesh,
    out_type=jax.ShapeDtypeStruct((_BATCH, _EMB_DIM), jnp.float32),
    scratch_types=[
        pltpu.VMEM((_B_PER_W,), jnp.int32),
        pltpu.VMEM((_RING, _EMB_DIM, _LANES), jnp.float32),
        pltpu.VMEM((_B_PER_W, _EMB_DIM), jnp.float32),
    ] + [pltpu.SemaphoreType.DMA] * _RING,
    compiler_params=pltpu.CompilerParams(needs_layout_passes=False),
)
def _gather_kernel(idx_hbm, table_hbm, out_hbm, idx_v, cols_v, rows_v, *sems):
    wid = lax.axis_index("s") * _NUM_CORES + lax.axis_index("c")
    base = wid * _B_PER_W
    pltpu.sync_copy(idx_hbm.at[pl.ds(base, _B_PER_W)], idx_v)

    c_lo = lax.iota(jnp.int32, 16)
    c_hi = c_lo + jnp.int32(16)

    def fire(slot, t):
        tc = lax.shift_right_logical(t, 7)
        off = pl.multiple_of(tc * _LANES, _LANES)
        pltpu.async_copy(
            table_hbm.at[:, pl.ds(off, _LANES)], cols_v.at[slot], sems[slot])

    def drain_extract(slot, t, dst_i):
        pltpu.make_async_copy(
            table_hbm.at[:, pl.ds(0, _LANES)], cols_v.at[slot],
            sems[slot]).wait()
        lane = lax.bitwise_and(t, jnp.int32(_LANES - 1))
        j = jnp.full((16,), lane, jnp.int32)
        d = jnp.full((16,), dst_i, jnp.int32)
        v0 = plsc.load_gather(cols_v.at[slot], [c_lo, j])
        v1 = plsc.load_gather(cols_v.at[slot], [c_hi, j])
        plsc.store_scatter(rows_v, [d, c_lo], v0)
        plsc.store_scatter(rows_v, [d, c_hi], v1)

    def body(t, carry):
        vec = idx_v[pl.ds(t * 16, 16)]
        for half in range(2):
            for l in range(_RING):
                fire(l, vec[half * _RING + l])
            for l in range(_RING):
                drain_extract(l, vec[half * _RING + l],
                              t * 16 + half * _RING + l)
        return carry

    lax.fori_loop(0, _B_PER_W // 16, body, 0)

    pltpu.sync_copy(rows_v, out_hbm.at[pl.ds(base, _B_PER_W)])


def kernel(targets, embs):
    return _gather_kernel(targets.astype(jnp.int32), embs.T)
